# baseline (speedup 1.0000x reference)
import functools

import jax
import jax.numpy as jnp
import numpy as np
from jax import lax
from jax.experimental import pallas as pl
from jax.experimental.pallas import tpu as pltpu
from jax.experimental.pallas import tpu_sc as plsc

_LEVELS = (8, 8, 8, 6, 5)
_CB = len(_LEVELS)
_CBP = 8
_BASIS = tuple(np.cumprod((1,) + _LEVELS[:-1]).astype(np.float32).tolist())
_HALF_WIDTH = tuple(float(l // 2) for l in _LEVELS)
_SCALE = tuple(float(l if l % 2 == 0 else l - 1) for l in _LEVELS)
_NBUF = 4

_NC, _NS = 2, 16
_NW = _NC * _NS
_LANES = 16


def _compute_one(z_blk, wi, bi, wo, bo):
    p = jax.lax.dot_general(
        wi, z_blk, (((1,), (0,)), ((), ())),
        preferred_element_type=jnp.float32,
        precision=jax.lax.Precision.DEFAULT)
    p = p + bi

    rows = jax.lax.broadcasted_iota(jnp.int32, p.shape, 0)
    scale = jnp.zeros_like(p)
    lmax = jnp.zeros_like(p)
    for c in range(_CB):
        scale = jnp.where(rows == c, _SCALE[c], scale)
        lmax = jnp.where(rows == c, float(_LEVELS[c] - 1), lmax)
    t = (p + 0.5) * scale
    idx_f = jnp.clip(jnp.ceil(t - 0.5), 0.0, lmax)
    q = idx_f / jnp.where(scale == 0.0, 1.0, scale) - 0.5
    codes = jnp.where(rows < _CB, p + (q - p), 0.0)

    out = jax.lax.dot_general(
        wo, codes, (((1,), (0,)), ((), ())),
        preferred_element_type=jnp.float32,
        precision=jax.lax.Precision.DEFAULT)
    out = out + bo

    diff = z_blk - out
    part = jnp.sum(diff * diff)
    return p, out, part


def _lq_kernel(z_hbm, wi_ref, bi_ref, wo_ref, bo_ref,
               out_hbm, p_ref, loss_ref,
               in_buf, out_buf, in_sems, out_sems, *, nb, n_tokens, d):
    def start_in(b):
        pltpu.make_async_copy(
            z_hbm.at[b], in_buf.at[b % _NBUF], in_sems.at[b % _NBUF]).start()

    def wait_in(b):
        pltpu.make_async_copy(
            z_hbm.at[b], in_buf.at[b % _NBUF], in_sems.at[b % _NBUF]).wait()

    def start_out(b):
        pltpu.make_async_copy(
            out_buf.at[b % _NBUF], out_hbm.at[b], out_sems.at[b % _NBUF]).start()

    def wait_out(b):
        pltpu.make_async_copy(
            out_buf.at[b % _NBUF], out_hbm.at[b], out_sems.at[b % _NBUF]).wait()

    for b in range(min(_NBUF, nb)):
        start_in(b)

    loss_acc = jnp.float32(0.0)
    for b in range(nb):
        wait_in(b)
        p, out, part = _compute_one(
            in_buf[b % _NBUF], wi_ref[...], bi_ref[...], wo_ref[...],
            bo_ref[...])
        loss_acc = loss_acc + part
        p_ref[b] = p
        if b >= _NBUF:
            wait_out(b - _NBUF)
        out_buf[b % _NBUF] = out
        start_out(b)
        if b + _NBUF < nb:
            start_in(b + _NBUF)

    for b in range(max(0, nb - _NBUF), nb):
        wait_out(b)

    loss_ref[...] = (loss_acc * (0.2 / (n_tokens * d))).reshape(1, 1)


def _sc_indices_kernel(p_hbm, idx_hbm, p_v, idx_v, *, tok_per_w):
    wid = lax.axis_index("s") * _NC + lax.axis_index("c")
    bi = wid // 2
    half = wid % 2
    base = half * tok_per_w
    pltpu.sync_copy(p_hbm.at[bi, :, pl.ds(base, tok_per_w)], p_v)

    for g in range(tok_per_w // _LANES):
        sl = pl.ds(g * _LANES, _LANES)
        acc = None
        for c in range(_CB):
            x = p_v[c, sl]
            t = (x + 0.5) * _SCALE[c]
            tc = jnp.minimum(jnp.maximum(t, 0.0), float(_LEVELS[c] - 1))
            iv = (tc + 0.5).astype(jnp.int32).astype(jnp.float32)
            iv = iv - jnp.where((iv - tc) == 0.5, 1.0, 0.0)
            q = iv / _SCALE[c] - 0.5
            scaled = q * (2.0 * _HALF_WIDTH[c]) + _HALF_WIDTH[c]
            term = scaled * _BASIS[c]
            acc = term if acc is None else acc + term
        idx_v[sl] = acc.astype(jnp.int32)

    pltpu.sync_copy(idx_v, idx_hbm.at[pl.ds(wid * tok_per_w, tok_per_w)])


@jax.jit
def kernel(z, W_in, b_in, W_out, b_out, v0, v1, v2, v3, v4):
    b, d, h, w = z.shape
    n = h * w
    z3 = z.reshape(b, d, n)

    wi = jnp.zeros((_CBP, d), jnp.float32).at[:_CB].set(W_in.T)
    bi = jnp.zeros((_CBP, 1), jnp.float32).at[:_CB, 0].set(b_in)
    wo = jnp.zeros((d, _CBP), jnp.float32).at[:, :_CB].set(W_out.T)
    bo = b_out.reshape(d, 1)

    out3, p_all, loss = pl.pallas_call(
        functools.partial(_lq_kernel, nb=b, n_tokens=b * n, d=d),
        in_specs=[
            pl.BlockSpec(memory_space=pl.ANY),
            pl.BlockSpec(memory_space=pltpu.VMEM),
            pl.BlockSpec(memory_space=pltpu.VMEM),
            pl.BlockSpec(memory_space=pltpu.VMEM),
            pl.BlockSpec(memory_space=pltpu.VMEM),
        ],
        out_specs=[
            pl.BlockSpec(memory_space=pl.ANY),
            pl.BlockSpec(memory_space=pltpu.VMEM),
            pl.BlockSpec(memory_space=pltpu.VMEM),
        ],
        out_shape=[
            jax.ShapeDtypeStruct((b, d, n), jnp.float32),
            jax.ShapeDtypeStruct((b, _CBP, n), jnp.float32),
            jax.ShapeDtypeStruct((1, 1), jnp.float32),
        ],
        scratch_shapes=[
            pltpu.VMEM((_NBUF, d, n), jnp.float32),
            pltpu.VMEM((_NBUF, d, n), jnp.float32),
            pltpu.SemaphoreType.DMA((_NBUF,)),
            pltpu.SemaphoreType.DMA((_NBUF,)),
        ],
    )(z3, wi, bi, wo, bo)

    tok_per_w = (b * n) // _NW
    mesh = plsc.VectorSubcoreMesh(core_axis_name="c", subcore_axis_name="s")
    idx_flat = pl.kernel(
        functools.partial(_sc_indices_kernel, tok_per_w=tok_per_w),
        out_type=jax.ShapeDtypeStruct((b * n,), jnp.int32),
        mesh=mesh,
        scratch_types=[
            pltpu.VMEM((_CBP, tok_per_w), jnp.float32),
            pltpu.VMEM((tok_per_w,), jnp.int32),
        ],
    )(p_all)

    out = out3.reshape(b, d, h, w)
    indices = idx_flat.reshape(b, h, w)
    return out, indices, loss[0, 0]


# speedup vs baseline: 1.1228x; 1.1228x over previous
import functools

import jax
import jax.numpy as jnp
import numpy as np
from jax.experimental import pallas as pl
from jax.experimental.pallas import tpu as pltpu

_LEVELS = (8, 8, 8, 6, 5)
_CB = len(_LEVELS)
_CBP = 8
_BASIS = tuple(np.cumprod((1,) + _LEVELS[:-1]).astype(np.float32).tolist())
_HALF_WIDTH = tuple(float(l // 2) for l in _LEVELS)
_SCALE = tuple(float(l if l % 2 == 0 else l - 1) for l in _LEVELS)
_NBUF = 4


def _compute_one(z_blk, wi, bi, wo, bo):
    p = jax.lax.dot_general(
        wi, z_blk, (((1,), (0,)), ((), ())),
        preferred_element_type=jnp.float32,
        precision=jax.lax.Precision.DEFAULT)
    p = p + bi

    rows = jax.lax.broadcasted_iota(jnp.int32, p.shape, 0)
    scale = jnp.zeros_like(p)
    lmax = jnp.zeros_like(p)
    for c in range(_CB):
        scale = jnp.where(rows == c, _SCALE[c], scale)
        lmax = jnp.where(rows == c, float(_LEVELS[c] - 1), lmax)
    t = (p + 0.5) * scale
    idx_f = jnp.clip(jnp.ceil(t - 0.5), 0.0, lmax)
    q = idx_f / jnp.where(scale == 0.0, 1.0, scale) - 0.5
    codes = jnp.where(rows < _CB, p + (q - p), 0.0)

    basis = jnp.zeros_like(p)
    hw = jnp.zeros_like(p)
    for c in range(_CB):
        basis = jnp.where(rows == c, _BASIS[c], basis)
        hw = jnp.where(rows == c, _HALF_WIDTH[c], hw)
    scaled = q * (2.0 * hw) + hw
    idx_sum = jnp.sum(jnp.where(rows < _CB, scaled * basis, 0.0), axis=0,
                      keepdims=True).astype(jnp.int32)

    out = jax.lax.dot_general(
        wo, codes, (((1,), (0,)), ((), ())),
        preferred_element_type=jnp.float32,
        precision=jax.lax.Precision.DEFAULT)
    out = out + bo

    diff = z_blk - out
    part = jnp.sum(diff * diff)
    return out, idx_sum, part


def _lq_kernel(z_hbm, wi_ref, bi_ref, wo_ref, bo_ref,
               out_hbm, idx_ref, loss_ref,
               in_buf, out_buf, in_sems, out_sems, *, nb, n_tokens, d):
    def start_in(b):
        pltpu.make_async_copy(
            z_hbm.at[b], in_buf.at[b % _NBUF], in_sems.at[b % _NBUF]).start()

    def wait_in(b):
        pltpu.make_async_copy(
            z_hbm.at[b], in_buf.at[b % _NBUF], in_sems.at[b % _NBUF]).wait()

    def start_out(b):
        pltpu.make_async_copy(
            out_buf.at[b % _NBUF], out_hbm.at[b], out_sems.at[b % _NBUF]).start()

    def wait_out(b):
        pltpu.make_async_copy(
            out_buf.at[b % _NBUF], out_hbm.at[b], out_sems.at[b % _NBUF]).wait()

    for b in range(min(_NBUF, nb)):
        start_in(b)

    loss_acc = jnp.float32(0.0)
    for b in range(nb):
        wait_in(b)
        out, idx_sum, part = _compute_one(
            in_buf[b % _NBUF], wi_ref[...], bi_ref[...], wo_ref[...],
            bo_ref[...])
        loss_acc = loss_acc + part
        idx_ref[b] = idx_sum
        if b >= _NBUF:
            wait_out(b - _NBUF)
        out_buf[b % _NBUF] = out
        start_out(b)
        if b + _NBUF < nb:
            start_in(b + _NBUF)

    for b in range(max(0, nb - _NBUF), nb):
        wait_out(b)

    loss_ref[...] = (loss_acc * (0.2 / (n_tokens * d))).reshape(1, 1)


@jax.jit
def kernel(z, W_in, b_in, W_out, b_out, v0, v1, v2, v3, v4):
    b, d, h, w = z.shape
    n = h * w
    z3 = z.reshape(b, d, n)

    wi = jnp.zeros((_CBP, d), jnp.float32).at[:_CB].set(W_in.T)
    bi = jnp.zeros((_CBP, 1), jnp.float32).at[:_CB, 0].set(b_in)
    wo = jnp.zeros((d, _CBP), jnp.float32).at[:, :_CB].set(W_out.T)
    bo = b_out.reshape(d, 1)

    out3, idx2, loss = pl.pallas_call(
        functools.partial(_lq_kernel, nb=b, n_tokens=b * n, d=d),
        in_specs=[
            pl.BlockSpec(memory_space=pl.ANY),
            pl.BlockSpec(memory_space=pltpu.VMEM),
            pl.BlockSpec(memory_space=pltpu.VMEM),
            pl.BlockSpec(memory_space=pltpu.VMEM),
            pl.BlockSpec(memory_space=pltpu.VMEM),
        ],
        out_specs=[
            pl.BlockSpec(memory_space=pl.ANY),
            pl.BlockSpec(memory_space=pltpu.VMEM),
            pl.BlockSpec(memory_space=pltpu.VMEM),
        ],
        out_shape=[
            jax.ShapeDtypeStruct((b, d, n), jnp.float32),
            jax.ShapeDtypeStruct((b, 1, n), jnp.int32),
            jax.ShapeDtypeStruct((1, 1), jnp.float32),
        ],
        scratch_shapes=[
            pltpu.VMEM((_NBUF, d, n), jnp.float32),
            pltpu.VMEM((_NBUF, d, n), jnp.float32),
            pltpu.SemaphoreType.DMA((_NBUF,)),
            pltpu.SemaphoreType.DMA((_NBUF,)),
        ],
    )(z3, wi, bi, wo, bo)

    out = out3.reshape(b, d, h, w)
    indices = idx2.reshape(b, h, w)
    return out, indices, loss[0, 0]
